# baseline (speedup 1.0000x reference)
import functools

import jax
import jax.numpy as jnp
from jax import lax
from jax.experimental import pallas as pl
from jax.experimental.pallas import tpu as pltpu
from jax.experimental.pallas import tpu_sc as plsc

N = 10000
E = 320000
IN_DIM = 128
HIDDEN = 16
NUM_CLASSES = 40

D = 48
NC = 1
NS = 16
NW = NC * NS
BLK = 128
BPW = 160
E_PAD = NW * BPW * BLK
ROWS = 10112
RPT = ROWS // NS
NBUF = 8
LEAD = 4

_MESH = plsc.VectorSubcoreMesh(
    core_axis_name="c", subcore_axis_name="s", num_cores=NC, num_subcores=NS
)


def _mlp_body(x_ref, w1_ref, w2_ref, o_ref):
    h = jnp.maximum(
        jnp.dot(x_ref[...], w1_ref[...], preferred_element_type=jnp.float32), 0.0
    )
    o_ref[...] = jnp.dot(h, w2_ref[...], preferred_element_type=jnp.float32)


def _mlp(x, w1, w2p):
    rb = 1000
    return pl.pallas_call(
        _mlp_body,
        grid=(N // rb,),
        in_specs=[
            pl.BlockSpec((rb, IN_DIM), lambda i: (i, 0)),
            pl.BlockSpec((IN_DIM, HIDDEN), lambda i: (0, 0)),
            pl.BlockSpec((HIDDEN, D), lambda i: (0, 0)),
        ],
        out_specs=pl.BlockSpec((rb, D), lambda i: (i, 0)),
        out_shape=jax.ShapeDtypeStruct((N, D), jnp.float32),
    )(x, w1, w2p)


def _combine_body(p_ref, o_ref):
    o_ref[...] = p_ref[...].sum(axis=0)


def _combine(part):
    rb = 1000
    return pl.pallas_call(
        _combine_body,
        grid=(N // rb,),
        in_specs=[pl.BlockSpec((NC, rb, D), lambda i: (0, i, 0))],
        out_specs=pl.BlockSpec((rb, D), lambda i: (i, 0)),
        out_shape=jax.ShapeDtypeStruct((N, D), jnp.float32),
    )(part)


def _sc_round_body(
    table, srcp, dstp, zeros_in, part, idx_s, idx_d, rows, acc, gsem, ssem
):
    cid = lax.axis_index("c")
    sid = lax.axis_index("s")
    wid = sid * NC + cid

    r0 = sid * RPT
    pltpu.sync_copy(zeros_in.at[pl.ds(r0, RPT)], acc.at[pl.ds(r0, RPT)])

    base = wid * BPW
    pltpu.sync_copy(srcp.at[pl.ds(base, BPW)], idx_s)
    pltpu.sync_copy(dstp.at[pl.ds(base, BPW)], idx_d)
    plsc.subcore_barrier()

    for b in range(LEAD):
        pltpu.async_copy(table.at[idx_s.at[b]], rows.at[b], gsem.at[b])

    def grp(g, carry):
        for b in range(NBUF):
            j = g * NBUF + b
            d = (b + LEAD) % NBUF

            @pl.when(j >= LEAD)
            def _():
                pltpu.make_async_copy(
                    rows.at[d], acc.at[idx_d.at[0]], ssem.at[d]
                ).wait()

            @pl.when(j + LEAD < BPW)
            def _():
                pltpu.async_copy(table.at[idx_s.at[j + LEAD]], rows.at[d], gsem.at[d])

            pltpu.make_async_copy(table.at[idx_s.at[0]], rows.at[b], gsem.at[b]).wait()
            pltpu.async_copy(rows.at[b], acc.at[idx_d.at[j]], ssem.at[b], add=True)

        return carry

    lax.fori_loop(0, BPW // NBUF, grp, 0)

    for x in range(BPW - LEAD, BPW):
        b = x % NBUF
        pltpu.make_async_copy(rows.at[b], acc.at[idx_d.at[0]], ssem.at[b]).wait()
    plsc.subcore_barrier()

    pltpu.sync_copy(acc.at[pl.ds(r0, RPT)], part.at[cid].at[pl.ds(r0, RPT)])


_sc_round = pl.kernel(
    _sc_round_body,
    out_type=jax.ShapeDtypeStruct((NC, ROWS, D), jnp.float32),
    mesh=_MESH,
    scratch_types=[
        pltpu.VMEM((BPW, BLK), jnp.int32),
        pltpu.VMEM((BPW, BLK), jnp.int32),
        pltpu.VMEM((NBUF, BLK, D), jnp.float32),
        pltpu.VMEM_SHARED((ROWS, D), jnp.float32),
        pltpu.SemaphoreType.DMA((NBUF,)),
        pltpu.SemaphoreType.DMA((NBUF,)),
    ],
    compiler_params=pltpu.CompilerParams(use_tc_tiling_on_sc=False),
)


def kernel(features, edge_index, W1, W2):
    w2p = jnp.pad(W2, ((0, 0), (0, D - NUM_CLASSES)))
    h = _mlp(features, W1, w2p)

    pad = E_PAD - E
    srcp = jnp.concatenate(
        [edge_index[0], jnp.zeros((pad,), jnp.int32)]
    ).reshape(E_PAD // BLK, BLK)
    dstp = jnp.concatenate(
        [edge_index[1], jnp.full((pad,), N, jnp.int32)]
    ).reshape(E_PAD // BLK, BLK)
    zeros_in = jnp.zeros((ROWS, D), jnp.float32)

    for _ in range(2):
        part = _sc_round(h, srcp, dstp, zeros_in)
        h = _combine(part)
    return h[:, :NUM_CLASSES]


# speedup vs baseline: 2.3650x; 2.3650x over previous
import functools

import jax
import jax.numpy as jnp
from jax import lax
from jax.experimental import pallas as pl
from jax.experimental.pallas import tpu as pltpu
from jax.experimental.pallas import tpu_sc as plsc

N = 10000
E = 320000
IN_DIM = 128
HIDDEN = 16
NUM_CLASSES = 40

D = 48
NC = 2
NS = 16
NW = NC * NS
BLK = 128
BPW = 80
E_PAD = NW * BPW * BLK
ROWS = 10112
RPT = ROWS // NS
NBUF = 8
LEAD = 4

_MESH = plsc.VectorSubcoreMesh(
    core_axis_name="c", subcore_axis_name="s", num_cores=NC, num_subcores=NS
)


def _mlp_body(x_ref, w1_ref, w2_ref, o_ref):
    h = jnp.maximum(
        jnp.dot(x_ref[...], w1_ref[...], preferred_element_type=jnp.float32), 0.0
    )
    o_ref[...] = jnp.dot(h, w2_ref[...], preferred_element_type=jnp.float32)


def _mlp(x, w1, w2p):
    rb = RPT
    return pl.pallas_call(
        _mlp_body,
        grid=(ROWS // rb,),
        in_specs=[
            pl.BlockSpec((rb, IN_DIM), lambda i: (i, 0)),
            pl.BlockSpec((IN_DIM, HIDDEN), lambda i: (0, 0)),
            pl.BlockSpec((HIDDEN, D), lambda i: (0, 0)),
        ],
        out_specs=pl.BlockSpec((rb, D), lambda i: (i, 0)),
        out_shape=jax.ShapeDtypeStruct((ROWS, D), jnp.float32),
    )(x, w1, w2p)


def _combine_body(p_ref, o_ref):
    o_ref[...] = p_ref[0] + p_ref[1]


def _combine(part):
    rb = RPT
    return pl.pallas_call(
        _combine_body,
        grid=(ROWS // rb,),
        in_specs=[pl.BlockSpec((2, rb, D), lambda i: (0, i, 0))],
        out_specs=pl.BlockSpec((rb, D), lambda i: (i, 0)),
        out_shape=jax.ShapeDtypeStruct((ROWS, D), jnp.float32),
    )(part)


def _sc_round_body(
    table, srcp, dstp, zeros_in, part, idx_s, idx_d, rows, tbl, acc, gsem, ssem
):
    cid = lax.axis_index("c")
    sid = lax.axis_index("s")
    wid = sid * NC + cid

    r0 = sid * RPT
    pltpu.sync_copy(zeros_in.at[pl.ds(r0, RPT)], acc.at[pl.ds(r0, RPT)])
    pltpu.sync_copy(table.at[pl.ds(r0, RPT)], tbl.at[pl.ds(r0, RPT)])

    base = wid * BPW
    pltpu.sync_copy(srcp.at[pl.ds(base, BPW)], idx_s)
    pltpu.sync_copy(dstp.at[pl.ds(base, BPW)], idx_d)
    plsc.subcore_barrier()

    for b in range(LEAD):
        pltpu.async_copy(tbl.at[idx_s.at[b]], rows.at[b], gsem.at[b])

    def grp(g, carry):
        for b in range(NBUF):
            j = g * NBUF + b
            d = (b + LEAD) % NBUF

            @pl.when(j >= LEAD)
            def _():
                pltpu.make_async_copy(
                    rows.at[d], acc.at[idx_d.at[0]], ssem.at[d]
                ).wait()

            @pl.when(j + LEAD < BPW)
            def _():
                pltpu.async_copy(tbl.at[idx_s.at[j + LEAD]], rows.at[d], gsem.at[d])

            pltpu.make_async_copy(tbl.at[idx_s.at[0]], rows.at[b], gsem.at[b]).wait()
            pltpu.async_copy(rows.at[b], acc.at[idx_d.at[j]], ssem.at[b], add=True)

        return carry

    lax.fori_loop(0, BPW // NBUF, grp, 0)

    for x in range(BPW - LEAD, BPW):
        b = x % NBUF
        pltpu.make_async_copy(rows.at[b], acc.at[idx_d.at[0]], ssem.at[b]).wait()
    plsc.subcore_barrier()

    pltpu.sync_copy(acc.at[pl.ds(r0, RPT)], part.at[cid].at[pl.ds(r0, RPT)])


_sc_round = pl.kernel(
    _sc_round_body,
    out_type=jax.ShapeDtypeStruct((NC, ROWS, D), jnp.float32),
    mesh=_MESH,
    scratch_types=[
        pltpu.VMEM((BPW, BLK), jnp.int32),
        pltpu.VMEM((BPW, BLK), jnp.int32),
        pltpu.VMEM((NBUF, BLK, D), jnp.float32),
        pltpu.VMEM_SHARED((ROWS, D), jnp.float32),
        pltpu.VMEM_SHARED((ROWS, D), jnp.float32),
        pltpu.SemaphoreType.DMA((NBUF,)),
        pltpu.SemaphoreType.DMA((NBUF,)),
    ],
    compiler_params=pltpu.CompilerParams(use_tc_tiling_on_sc=False),
)


def kernel(features, edge_index, W1, W2):
    w2p = jnp.pad(W2, ((0, 0), (0, D - NUM_CLASSES)))
    featp = jnp.pad(features, ((0, ROWS - N), (0, 0)))
    h = _mlp(featp, W1, w2p)

    pad = E_PAD - E
    srcp = jnp.concatenate(
        [edge_index[0], jnp.zeros((pad,), jnp.int32)]
    ).reshape(E_PAD // BLK, BLK)
    dstp = jnp.concatenate(
        [edge_index[1], jnp.full((pad,), N, jnp.int32)]
    ).reshape(E_PAD // BLK, BLK)
    zeros_in = jnp.zeros((ROWS, D), jnp.float32)

    for _ in range(2):
        part = _sc_round(h, srcp, dstp, zeros_in)
        h = _combine(part)
    return h[:N, :NUM_CLASSES]


# speedup vs baseline: 2.5006x; 1.0573x over previous
import jax
import jax.numpy as jnp
from jax import lax
from jax.experimental import pallas as pl
from jax.experimental.pallas import tpu as pltpu
from jax.experimental.pallas import tpu_sc as plsc

N = 10000
E = 320000
IN_DIM = 128
HIDDEN = 16
NUM_CLASSES = 40

D = 48
NC = 2
NS = 16
NW = NC * NS
BLK = 128
BPW = 80
E_PAD = NW * BPW * BLK
ROWS = 10112
RPT = ROWS // NS
NBUF = 8
LEAD = 4

_MESH = plsc.VectorSubcoreMesh(
    core_axis_name="c", subcore_axis_name="s", num_cores=NC, num_subcores=NS
)


def _mlp_body(x_ref, w1_ref, w2_ref, o_ref):
    h = jnp.maximum(
        jnp.dot(x_ref[...], w1_ref[...], preferred_element_type=jnp.float32), 0.0
    )
    o_ref[...] = jnp.dot(h, w2_ref[...], preferred_element_type=jnp.float32)


def _mlp(x, w1, w2p):
    rb = 1000
    return pl.pallas_call(
        _mlp_body,
        grid=(N // rb,),
        in_specs=[
            pl.BlockSpec((rb, IN_DIM), lambda i: (i, 0)),
            pl.BlockSpec((IN_DIM, HIDDEN), lambda i: (0, 0)),
            pl.BlockSpec((HIDDEN, D), lambda i: (0, 0)),
        ],
        out_specs=pl.BlockSpec((rb, D), lambda i: (i, 0)),
        out_shape=jax.ShapeDtypeStruct((ROWS, D), jnp.float32),
    )(x, w1, w2p)


def _combine_body(p_ref, o_ref):
    o_ref[...] = p_ref[0] + p_ref[1]


def _combine(part):
    rb = 1000
    return pl.pallas_call(
        _combine_body,
        grid=(N // rb,),
        in_specs=[pl.BlockSpec((2, rb, D), lambda i: (0, i, 0))],
        out_specs=pl.BlockSpec((rb, D), lambda i: (i, 0)),
        out_shape=jax.ShapeDtypeStruct((ROWS, D), jnp.float32),
    )(part)


def _final_body(p_ref, o_ref):
    s = p_ref[0] + p_ref[1]
    o_ref[...] = s[:, :NUM_CLASSES]


def _final(part):
    rb = 1000
    return pl.pallas_call(
        _final_body,
        grid=(N // rb,),
        in_specs=[pl.BlockSpec((2, rb, D), lambda i: (0, i, 0))],
        out_specs=pl.BlockSpec((rb, NUM_CLASSES), lambda i: (i, 0)),
        out_shape=jax.ShapeDtypeStruct((N, NUM_CLASSES), jnp.float32),
    )(part)


def _sc_round_body(
    table, srcp, dstp, zeros_in, part, idx_s, idx_d, rows, tbl, acc, gsem, ssem
):
    cid = lax.axis_index("c")
    sid = lax.axis_index("s")
    wid = sid * NC + cid

    r0 = sid * RPT
    pltpu.sync_copy(zeros_in.at[pl.ds(r0, RPT)], acc.at[pl.ds(r0, RPT)])
    pltpu.sync_copy(table.at[pl.ds(r0, RPT)], tbl.at[pl.ds(r0, RPT)])

    base = wid * BPW
    pltpu.sync_copy(srcp.at[pl.ds(base, BPW)], idx_s)
    pltpu.sync_copy(dstp.at[pl.ds(base, BPW)], idx_d)
    plsc.subcore_barrier()

    for b in range(LEAD):
        pltpu.async_copy(tbl.at[idx_s.at[b]], rows.at[b], gsem.at[b])

    def grp(g, carry):
        for b in range(NBUF):
            j = g * NBUF + b
            d = (b + LEAD) % NBUF

            @pl.when(j >= LEAD)
            def _():
                pltpu.make_async_copy(
                    rows.at[d], acc.at[idx_d.at[0]], ssem.at[d]
                ).wait()

            @pl.when(j + LEAD < BPW)
            def _():
                pltpu.async_copy(tbl.at[idx_s.at[j + LEAD]], rows.at[d], gsem.at[d])

            pltpu.make_async_copy(tbl.at[idx_s.at[0]], rows.at[b], gsem.at[b]).wait()
            pltpu.async_copy(rows.at[b], acc.at[idx_d.at[j]], ssem.at[b], add=True)

        return carry

    lax.fori_loop(0, BPW // NBUF, grp, 0)

    for x in range(BPW - LEAD, BPW):
        b = x % NBUF
        pltpu.make_async_copy(rows.at[b], acc.at[idx_d.at[0]], ssem.at[b]).wait()
    plsc.subcore_barrier()

    pltpu.sync_copy(acc.at[pl.ds(r0, RPT)], part.at[cid].at[pl.ds(r0, RPT)])


_sc_round = pl.kernel(
    _sc_round_body,
    out_type=jax.ShapeDtypeStruct((NC, ROWS, D), jnp.float32),
    mesh=_MESH,
    scratch_types=[
        pltpu.VMEM((BPW, BLK), jnp.int32),
        pltpu.VMEM((BPW, BLK), jnp.int32),
        pltpu.VMEM((NBUF, BLK, D), jnp.float32),
        pltpu.VMEM_SHARED((ROWS, D), jnp.float32),
        pltpu.VMEM_SHARED((ROWS, D), jnp.float32),
        pltpu.SemaphoreType.DMA((NBUF,)),
        pltpu.SemaphoreType.DMA((NBUF,)),
    ],
    compiler_params=pltpu.CompilerParams(use_tc_tiling_on_sc=False),
)


def kernel(features, edge_index, W1, W2):
    w2p = jnp.pad(W2, ((0, 0), (0, D - NUM_CLASSES)))
    h = _mlp(features, W1, w2p)

    pad = E_PAD - E
    srcp = jnp.concatenate(
        [edge_index[0], jnp.zeros((pad,), jnp.int32)]
    ).reshape(E_PAD // BLK, BLK)
    dstp = jnp.concatenate(
        [edge_index[1], jnp.full((pad,), N, jnp.int32)]
    ).reshape(E_PAD // BLK, BLK)
    zeros_in = jnp.zeros((ROWS, D), jnp.float32)

    part = _sc_round(h, srcp, dstp, zeros_in)
    h1 = _combine(part)
    part = _sc_round(h1, srcp, dstp, zeros_in)
    return _final(part)


# speedup vs baseline: 2.5398x; 1.0157x over previous
import jax
import jax.numpy as jnp
from jax import lax
from jax.experimental import pallas as pl
from jax.experimental.pallas import tpu as pltpu
from jax.experimental.pallas import tpu_sc as plsc

N = 10000
E = 320000
IN_DIM = 128
HIDDEN = 16
NUM_CLASSES = 40

D = 48
NC = 2
NS = 16
NW = NC * NS
BLK = 128
BPW = 80
E_PAD = NW * BPW * BLK
ROWS = 10112
RPT = ROWS // NS
NBUF = 8
LEAD = 4

_MESH = plsc.VectorSubcoreMesh(
    core_axis_name="c", subcore_axis_name="s", num_cores=NC, num_subcores=NS
)


def _mlp_body(x_ref, w1_ref, w2_ref, o_ref):
    h = jnp.maximum(
        jnp.dot(x_ref[...], w1_ref[...], preferred_element_type=jnp.float32), 0.0
    )
    o_ref[...] = jnp.dot(h, w2_ref[...], preferred_element_type=jnp.float32)


def _mlp(x, w1, w2p):
    rb = 2000
    return pl.pallas_call(
        _mlp_body,
        grid=(N // rb,),
        in_specs=[
            pl.BlockSpec((rb, IN_DIM), lambda i: (i, 0)),
            pl.BlockSpec((IN_DIM, HIDDEN), lambda i: (0, 0)),
            pl.BlockSpec((HIDDEN, D), lambda i: (0, 0)),
        ],
        out_specs=pl.BlockSpec((rb, D), lambda i: (i, 0)),
        out_shape=jax.ShapeDtypeStruct((ROWS, D), jnp.float32),
    )(x, w1, w2p)


def _combine_body(p_ref, o_ref):
    o_ref[...] = p_ref[0] + p_ref[1]


def _combine(part):
    rb = 1000
    return pl.pallas_call(
        _combine_body,
        grid=(N // rb,),
        in_specs=[pl.BlockSpec((2, rb, D), lambda i: (0, i, 0))],
        out_specs=pl.BlockSpec((rb, D), lambda i: (i, 0)),
        out_shape=jax.ShapeDtypeStruct((ROWS, D), jnp.float32),
    )(part)


def _final_body(p_ref, o_ref):
    s = p_ref[0] + p_ref[1]
    o_ref[...] = s[:, :NUM_CLASSES]


def _final(part):
    rb = 1000
    return pl.pallas_call(
        _final_body,
        grid=(N // rb,),
        in_specs=[pl.BlockSpec((2, rb, D), lambda i: (0, i, 0))],
        out_specs=pl.BlockSpec((rb, NUM_CLASSES), lambda i: (i, 0)),
        out_shape=jax.ShapeDtypeStruct((N, NUM_CLASSES), jnp.float32),
    )(part)


def _sc_round_body(
    table, srcp, dstp, zeros_in, part, idx_s, idx_d, rows, tbl, acc, gsem, ssem
):
    cid = lax.axis_index("c")
    sid = lax.axis_index("s")
    wid = sid * NC + cid

    r0 = sid * RPT
    pltpu.sync_copy(zeros_in.at[pl.ds(r0, RPT)], acc.at[pl.ds(r0, RPT)])
    pltpu.sync_copy(table.at[pl.ds(r0, RPT)], tbl.at[pl.ds(r0, RPT)])

    base = wid * BPW
    pltpu.sync_copy(srcp.at[pl.ds(base, BPW)], idx_s)
    pltpu.sync_copy(dstp.at[pl.ds(base, BPW)], idx_d)
    plsc.subcore_barrier()

    for b in range(LEAD):
        pltpu.async_copy(tbl.at[idx_s.at[b]], rows.at[b], gsem.at[b])

    def grp(g, carry):
        for b in range(NBUF):
            j = g * NBUF + b
            d = (b + LEAD) % NBUF

            @pl.when(j >= LEAD)
            def _():
                pltpu.make_async_copy(
                    rows.at[d], acc.at[idx_d.at[0]], ssem.at[d]
                ).wait()

            @pl.when(j + LEAD < BPW)
            def _():
                pltpu.async_copy(tbl.at[idx_s.at[j + LEAD]], rows.at[d], gsem.at[d])

            pltpu.make_async_copy(tbl.at[idx_s.at[0]], rows.at[b], gsem.at[b]).wait()
            pltpu.async_copy(rows.at[b], acc.at[idx_d.at[j]], ssem.at[b], add=True)

        return carry

    lax.fori_loop(0, BPW // NBUF, grp, 0)

    for x in range(BPW - LEAD, BPW):
        b = x % NBUF
        pltpu.make_async_copy(rows.at[b], acc.at[idx_d.at[0]], ssem.at[b]).wait()
    plsc.subcore_barrier()

    pltpu.sync_copy(acc.at[pl.ds(r0, RPT)], part.at[cid].at[pl.ds(r0, RPT)])


_sc_round = pl.kernel(
    _sc_round_body,
    out_type=jax.ShapeDtypeStruct((NC, ROWS, D), jnp.float32),
    mesh=_MESH,
    scratch_types=[
        pltpu.VMEM((BPW, BLK), jnp.int32),
        pltpu.VMEM((BPW, BLK), jnp.int32),
        pltpu.VMEM((NBUF, BLK, D), jnp.float32),
        pltpu.VMEM_SHARED((ROWS, D), jnp.float32),
        pltpu.VMEM_SHARED((ROWS, D), jnp.float32),
        pltpu.SemaphoreType.DMA((NBUF,)),
        pltpu.SemaphoreType.DMA((NBUF,)),
    ],
    compiler_params=pltpu.CompilerParams(use_tc_tiling_on_sc=False),
)


def kernel(features, edge_index, W1, W2):
    w2p = jnp.pad(W2, ((0, 0), (0, D - NUM_CLASSES)))
    h = _mlp(features, W1, w2p)

    pad = E_PAD - E
    srcp = jnp.concatenate(
        [edge_index[0], jnp.zeros((pad,), jnp.int32)]
    ).reshape(E_PAD // BLK, BLK)
    dstp = jnp.concatenate(
        [edge_index[1], jnp.full((pad,), N, jnp.int32)]
    ).reshape(E_PAD // BLK, BLK)
    zeros_in = jnp.zeros((ROWS, D), jnp.float32)

    part = _sc_round(h, srcp, dstp, zeros_in)
    h1 = _combine(part)
    part = _sc_round(h1, srcp, dstp, zeros_in)
    return _final(part)
